# baseline (speedup 1.0000x reference)
import numpy as np
import jax
import jax.numpy as jnp
from jax.experimental import pallas as pl
from jax.experimental.pallas import tpu as pltpu

_D_IN = 16
_N = 26
_SIZES = [_N - i for i in range(_N)]
_TOTAL = sum(_SIZES)
_OFFS = [int(v) for v in np.cumsum([0] + _SIZES)]
_PAD = 128
_BB = 512


def _copies(obufs, out_refs, step, parity):
    row0 = step * _BB
    return [
        pltpu.make_async_copy(
            obufs[parity * _N + i],
            out_refs[i].at[pl.ds(row0, _BB), :],
            obufs[2 * _N + parity],
        )
        for i in range(_N)
    ]


def _body(x_ref, w_ref, b_ref, *refs):
    out_refs = refs[:_N]
    obufs = refs[_N:]
    i = pl.program_id(0)
    nsteps = pl.num_programs(0)
    p = jax.lax.rem(i, 2)

    for sp in (0, 1):
        @pl.when((i >= 2) & (p == sp))
        def _(sp=sp):
            for c in _copies(obufs, out_refs, i - 2, sp):
                c.wait()

    x = x_ref[...]
    for h in range(_N):
        y = jax.lax.dot_general(
            x, w_ref[h], (((1,), (0,)), ((), ())),
            preferred_element_type=jnp.float32) + b_ref[h]
        for sp in (0, 1):
            @pl.when(p == sp)
            def _(y=y, sp=sp, h=h):
                dst = obufs[sp * _N + h]
                dst[...] = y[:, : dst.shape[1]]

    for sp in (0, 1):
        @pl.when(p == sp)
        def _(sp=sp):
            for c in _copies(obufs, out_refs, i, sp):
                c.start()

    last = nsteps - 1
    pl_last = last % 2 if isinstance(last, int) else None
    for sp in (0, 1):
        @pl.when((i == last) & (p == sp))
        def _(sp=sp):
            @pl.when(i >= 1)
            def _():
                for c in _copies(obufs, out_refs, i - 1, 1 - sp):
                    c.wait()

            for c in _copies(obufs, out_refs, i, sp):
                c.wait()


def kernel(x, W_cat, b_cat):
    B = x.shape[0]
    Wt = W_cat.T
    W_heads = jnp.stack([
        jnp.pad(Wt[:, _OFFS[i]:_OFFS[i + 1]], ((0, 0), (0, _PAD - _SIZES[i])))
        for i in range(_N)
    ])
    b_heads = jnp.stack([
        jnp.pad(b_cat[_OFFS[i]:_OFFS[i + 1]], (0, _PAD - _SIZES[i]))
        for i in range(_N)
    ])[:, None, :]

    grid = (B // _BB,)
    out_shapes = [
        jax.ShapeDtypeStruct((B, _SIZES[i]), jnp.float32) for i in range(_N)
    ]
    out_specs = [
        pl.BlockSpec(memory_space=pl.ANY) for _ in range(_N)
    ]
    in_specs = [
        pl.BlockSpec((_BB, _D_IN), lambda i: (i, 0)),
        pl.BlockSpec((_N, _D_IN, _PAD), lambda i: (0, 0, 0)),
        pl.BlockSpec((_N, 1, _PAD), lambda i: (0, 0, 0)),
    ]
    scratch_shapes = (
        [pltpu.VMEM((_BB, _SIZES[i]), jnp.float32) for i in range(_N)]
        + [pltpu.VMEM((_BB, _SIZES[i]), jnp.float32) for i in range(_N)]
        + [pltpu.SemaphoreType.DMA, pltpu.SemaphoreType.DMA]
    )
    outs = pl.pallas_call(
        _body,
        grid=grid,
        in_specs=in_specs,
        out_specs=out_specs,
        out_shape=out_shapes,
        scratch_shapes=scratch_shapes,
    )(x, W_heads, b_heads)
    return tuple(outs)


# speedup vs baseline: 1.3332x; 1.3332x over previous
import numpy as np
import jax
import jax.numpy as jnp
from jax.experimental import pallas as pl

_D_IN = 16
_N = 26
_SIZES = [_N - i for i in range(_N)]
_TOTAL = sum(_SIZES)
_OFFS = np.cumsum([0] + _SIZES)
_PAD = 128

_BB = 2048


def _body(x_ref, w_ref, b_ref, *out_refs):
    x = x_ref[...]
    for i in range(_N):
        y = jax.lax.dot_general(
            x, w_ref[i], (((1,), (0,)), ((), ())),
            preferred_element_type=jnp.float32)
        y = y + b_ref[i]
        out_refs[i][...] = y[:, : _SIZES[i]]


def kernel(x, W_cat, b_cat):
    B = x.shape[0]
    Wt = W_cat.T
    heads_w = [
        jnp.pad(Wt[:, _OFFS[i]:_OFFS[i + 1]], ((0, 0), (0, _PAD - _SIZES[i])))
        for i in range(_N)
    ]
    W_heads = jnp.stack(heads_w)
    heads_b = [
        jnp.pad(b_cat[_OFFS[i]:_OFFS[i + 1]], (0, _PAD - _SIZES[i]))
        for i in range(_N)
    ]
    b_heads = jnp.stack(heads_b)[:, None, :]

    grid = (B // _BB,)
    out_shapes = [
        jax.ShapeDtypeStruct((B, _SIZES[i]), jnp.float32) for i in range(_N)
    ]
    out_specs = [
        pl.BlockSpec((_BB, _SIZES[i]), lambda i: (i, 0)) for i in range(_N)
    ]
    in_specs = [
        pl.BlockSpec((_BB, _D_IN), lambda i: (i, 0)),
        pl.BlockSpec((_N, _D_IN, _PAD), lambda i: (0, 0, 0)),
        pl.BlockSpec((_N, 1, _PAD), lambda i: (0, 0, 0)),
    ]
    outs = pl.pallas_call(
        _body,
        grid=grid,
        in_specs=in_specs,
        out_specs=out_specs,
        out_shape=out_shapes,
    )(x, W_heads, b_heads)
    return tuple(outs)


# speedup vs baseline: 1.3386x; 1.0041x over previous
import numpy as np
import jax
import jax.numpy as jnp
from jax.experimental import pallas as pl

_D_IN = 16
_N = 26
_SIZES = [_N - i for i in range(_N)]
_TOTAL = sum(_SIZES)
_OFFS = np.cumsum([0] + _SIZES)
_PAD = 128

_BB = 1024


def _body(x_ref, w_ref, b_ref, *out_refs):
    x = x_ref[...]
    for i in range(_N):
        y = jax.lax.dot_general(
            x, w_ref[i], (((1,), (0,)), ((), ())),
            preferred_element_type=jnp.float32)
        y = y + b_ref[i]
        out_refs[i][...] = y[:, : _SIZES[i]]


def kernel(x, W_cat, b_cat):
    B = x.shape[0]
    Wt = W_cat.T
    heads_w = [
        jnp.pad(Wt[:, _OFFS[i]:_OFFS[i + 1]], ((0, 0), (0, _PAD - _SIZES[i])))
        for i in range(_N)
    ]
    W_heads = jnp.stack(heads_w)
    heads_b = [
        jnp.pad(b_cat[_OFFS[i]:_OFFS[i + 1]], (0, _PAD - _SIZES[i]))
        for i in range(_N)
    ]
    b_heads = jnp.stack(heads_b)[:, None, :]

    grid = (B // _BB,)
    out_shapes = [
        jax.ShapeDtypeStruct((B, _SIZES[i]), jnp.float32) for i in range(_N)
    ]
    out_specs = [
        pl.BlockSpec((_BB, _SIZES[i]), lambda i: (i, 0)) for i in range(_N)
    ]
    in_specs = [
        pl.BlockSpec((_BB, _D_IN), lambda i: (i, 0)),
        pl.BlockSpec((_N, _D_IN, _PAD), lambda i: (0, 0, 0)),
        pl.BlockSpec((_N, 1, _PAD), lambda i: (0, 0, 0)),
    ]
    outs = pl.pallas_call(
        _body,
        grid=grid,
        in_specs=in_specs,
        out_specs=out_specs,
        out_shape=out_shapes,
    )(x, W_heads, b_heads)
    return tuple(outs)
